# baseline (speedup 1.0000x reference)
import jax
import jax.numpy as jnp
from jax import lax
from jax.experimental import pallas as pl
from jax.experimental.pallas import tpu as pltpu, tpu_sc as plsc

_B = 16384
_F = 1024
_NCP = 11
_TAB = _F * _NCP
_LOC = 5.0
_MAXL = 9
_NW = 32
_R_SC = 8192
_RPW = _R_SC // _NW
_CIN = 32
_COUT = 16
_NCIN = _RPW // _CIN
_NCOUT = _RPW // _COUT
_L = 16
_NFB = _F // _L


def _sc_body(x_hbm, tab_hbm, out_hbm, tab_v, pk_v, x_v, o_v,
             sem_in0, sem_in1, sem_out0, sem_out1):
    wid = lax.axis_index("s") * 2 + lax.axis_index("c")
    row0 = wid * _RPW
    pltpu.sync_copy(tab_hbm, tab_v.at[pl.ds(0, _TAB)])
    iota = lax.iota(jnp.int32, _L)
    iota11 = iota * _NCP

    def tab_body(k, carry):
        p0 = k * _L
        tv = tab_v[pl.ds(p0, _L)]
        tv1 = tab_v[pl.ds(p0 + 1, _L)]
        ivec = jnp.mod(p0 + iota, _NCP).astype(jnp.float32)
        d = tv1 - tv
        a = tv - ivec * d
        pk = plsc.bitcast(plsc.pack(a, d, format=plsc.PackFormat.INTERLEAVED),
                          jnp.int32)
        pk_v[pl.ds(p0, _L)] = pk
        return carry

    lax.fori_loop(0, _TAB // _L, tab_body, 0)

    sems_in = (sem_in0, sem_in1)
    sems_out = (sem_out0, sem_out1)

    def start_in(c, b):
        pltpu.async_copy(x_hbm.at[pl.ds(row0 + c * _CIN, _CIN)],
                         x_v.at[b], sems_in[b]).start()

    def wait_in(b):
        pltpu.make_async_copy(x_hbm.at[pl.ds(0, _CIN)], x_v.at[b],
                              sems_in[b]).wait()

    def start_out(oh, ob):
        pltpu.async_copy(o_v.at[ob], out_hbm.at[pl.ds(row0 + oh * _COUT, _COUT)],
                         sems_out[ob]).start()

    def wait_out(ob):
        pltpu.make_async_copy(o_v.at[ob], out_hbm.at[pl.ds(0, _COUT)],
                              sems_out[ob]).wait()

    start_in(0, 0)
    start_in(1, 1)

    def compute_half(b, h, ob):
        def fb_body(j, carry):
            f0 = j * _L
            fbase = f0 * _NCP + iota11

            @plsc.parallel_loop(0, _COUT, unroll=4)
            def row_body(r):
                xv = x_v[b, h * _COUT + r, pl.ds(f0, _L)]
                scaled = xv + _LOC
                li = jnp.clip(scaled.astype(jnp.int32), 0, _MAXL)
                flat = fbase + li
                pk = plsc.load_gather(pk_v, [flat])
                ab = plsc.bitcast(pk, jnp.bfloat16)
                av, bv = plsc.unpack(ab, format=plsc.PackFormat.INTERLEAVED)
                o_v[ob, r, pl.ds(f0, _L)] = av + scaled * bv

            return carry

        lax.fori_loop(0, _NFB, fb_body, 0)

    def cc_body(cc, carry):
        for b in range(2):
            c = cc * 2 + b
            wait_in(b)
            for h in range(2):
                oh = 2 * c + h

                @pl.when(oh >= 2)
                def _():
                    wait_out(h)

                compute_half(b, h, h)
                start_out(oh, h)

            @pl.when(c + 2 < _NCIN)
            def _():
                start_in(c + 2, b)

        return carry

    lax.fori_loop(0, _NCIN // 2, cc_body, 0)
    wait_out(0)
    wait_out(1)


_sc_call = pl.kernel(
    _sc_body,
    out_type=jax.ShapeDtypeStruct((_R_SC, _F), jnp.float32),
    mesh=plsc.VectorSubcoreMesh(core_axis_name="c", subcore_axis_name="s"),
    compiler_params=pltpu.CompilerParams(needs_layout_passes=False),
    scratch_types=[
        pltpu.VMEM((_TAB + _L,), jnp.float32),
        pltpu.VMEM((_TAB,), jnp.int32),
        pltpu.VMEM((2, _CIN, _F), jnp.float32),
        pltpu.VMEM((2, _COUT, _F), jnp.float32),
        pltpu.SemaphoreType.DMA,
        pltpu.SemaphoreType.DMA,
        pltpu.SemaphoreType.DMA,
        pltpu.SemaphoreType.DMA,
    ],
)


_BR = 512
_N_TC = _B - _R_SC


def _tc_body(tt_ref, x_ref, o_ref):
    tt = tt_ref[...]
    x = x_ref[...]
    scaled = x + _LOC
    d = tt[1:_NCP] - tt[0:_NCP - 1]
    ii = lax.broadcasted_iota(jnp.int32, (_NCP - 1, _F), 0).astype(jnp.float32)
    a = tt[0:_NCP - 1] - ii * d
    acc_a = jnp.broadcast_to(a[0:1], x.shape)
    acc_b = jnp.broadcast_to(d[0:1], x.shape)
    for i in range(1, _NCP - 1):
        m = scaled >= float(i)
        acc_a = jnp.where(m, a[i:i + 1], acc_a)
        acc_b = jnp.where(m, d[i:i + 1], acc_b)
    o_ref[...] = acc_a + scaled * acc_b


_tc_call = pl.pallas_call(
    _tc_body,
    grid=(_N_TC // _BR,),
    in_specs=[
        pl.BlockSpec((_NCP, _F), lambda g: (0, 0)),
        pl.BlockSpec((_BR, _F), lambda g: (_R_SC // _BR + g, 0)),
    ],
    out_specs=pl.BlockSpec((_BR, _F), lambda g: (g, 0)),
    out_shape=jax.ShapeDtypeStruct((_N_TC, _F), jnp.float32),
)


def kernel(x, interp_tensor, feature_idx):
    del feature_idx
    tab = interp_tensor.reshape(-1)
    sc_out = _sc_call(x, tab)
    tt = interp_tensor.T + 0.0 * sc_out[0, 0]
    tc_out = _tc_call(tt, x)
    return jnp.concatenate([sc_out, tc_out], axis=0)


# speedup vs baseline: 1.3602x; 1.3602x over previous
import jax
import jax.numpy as jnp
from jax import lax
from jax.experimental import pallas as pl
from jax.experimental.pallas import tpu as pltpu, tpu_sc as plsc

_B = 16384
_F = 1024
_NCP = 11
_TAB = _F * _NCP
_LOC = 5.0
_MAXL = 9
_NW = 32
_R_SC = 8192
_RPW = _R_SC // _NW
_CIN = 32
_COUT = 16
_NCIN = _RPW // _CIN
_NCOUT = _RPW // _COUT
_L = 16
_NFB = _F // _L


def _sc_body(x_hbm, tab_hbm, out_hbm, tab_v, pk_v, x_v, o_v,
             sem_in0, sem_in1, sem_out0, sem_out1):
    wid = lax.axis_index("s") * 2 + lax.axis_index("c")
    row0 = wid * _RPW
    pltpu.sync_copy(tab_hbm, tab_v.at[pl.ds(0, _TAB)])
    iota = lax.iota(jnp.int32, _L)
    iota11 = iota * _NCP

    def tab_body(k, carry):
        p0 = k * _L
        tv = tab_v[pl.ds(p0, _L)]
        tv1 = tab_v[pl.ds(p0 + 1, _L)]
        ivec = jnp.mod(p0 + iota, _NCP).astype(jnp.float32)
        d = tv1 - tv
        a = tv - ivec * d
        pk = plsc.bitcast(plsc.pack(a, d, format=plsc.PackFormat.INTERLEAVED),
                          jnp.int32)
        pk_v[pl.ds(p0, _L)] = pk
        return carry

    lax.fori_loop(0, _TAB // _L, tab_body, 0)

    sems_in = (sem_in0, sem_in1)
    sems_out = (sem_out0, sem_out1)

    def start_in(c, b):
        pltpu.async_copy(x_hbm.at[pl.ds(row0 + c * _CIN, _CIN)],
                         x_v.at[b], sems_in[b]).start()

    def wait_in(b):
        pltpu.make_async_copy(x_hbm.at[pl.ds(0, _CIN)], x_v.at[b],
                              sems_in[b]).wait()

    def start_out(oh, ob):
        pltpu.async_copy(o_v.at[ob], out_hbm.at[pl.ds(row0 + oh * _COUT, _COUT)],
                         sems_out[ob]).start()

    def wait_out(ob):
        pltpu.make_async_copy(o_v.at[ob], out_hbm.at[pl.ds(0, _COUT)],
                              sems_out[ob]).wait()

    start_in(0, 0)
    start_in(1, 1)

    def compute_half(b, h, ob):
        def fb_body(j, carry):
            f0 = j * _L
            fbase = f0 * _NCP + iota11

            @plsc.parallel_loop(0, _COUT, unroll=4)
            def row_body(r):
                xv = x_v[b, h * _COUT + r, pl.ds(f0, _L)]
                scaled = xv + _LOC
                li = jnp.clip(scaled.astype(jnp.int32), 0, _MAXL)
                flat = fbase + li
                pk = plsc.load_gather(pk_v, [flat])
                ab = plsc.bitcast(pk, jnp.bfloat16)
                av, bv = plsc.unpack(ab, format=plsc.PackFormat.INTERLEAVED)
                o_v[ob, r, pl.ds(f0, _L)] = av + scaled * bv

            return carry

        lax.fori_loop(0, _NFB, fb_body, 0)

    def cc_body(cc, carry):
        for b in range(2):
            c = cc * 2 + b
            wait_in(b)
            for h in range(2):
                oh = 2 * c + h

                @pl.when(oh >= 2)
                def _():
                    wait_out(h)

                compute_half(b, h, h)
                start_out(oh, h)

            @pl.when(c + 2 < _NCIN)
            def _():
                start_in(c + 2, b)

        return carry

    lax.fori_loop(0, _NCIN // 2, cc_body, 0)
    wait_out(0)
    wait_out(1)


_sc_call = pl.kernel(
    _sc_body,
    out_type=jax.ShapeDtypeStruct((_B, _F), jnp.float32),
    mesh=plsc.VectorSubcoreMesh(core_axis_name="c", subcore_axis_name="s"),
    compiler_params=pltpu.CompilerParams(needs_layout_passes=False),
    scratch_types=[
        pltpu.VMEM((_TAB + _L,), jnp.float32),
        pltpu.VMEM((_TAB,), jnp.int32),
        pltpu.VMEM((2, _CIN, _F), jnp.float32),
        pltpu.VMEM((2, _COUT, _F), jnp.float32),
        pltpu.SemaphoreType.DMA,
        pltpu.SemaphoreType.DMA,
        pltpu.SemaphoreType.DMA,
        pltpu.SemaphoreType.DMA,
    ],
)


_BR = 512
_N_TC = _B - _R_SC


def _tc_body(sc_ref, tt_ref, x_ref, o_ref):
    del sc_ref
    tt = tt_ref[...]
    x = x_ref[...]
    scaled = x + _LOC
    d = tt[1:_NCP] - tt[0:_NCP - 1]
    ii = lax.broadcasted_iota(jnp.int32, (_NCP - 1, _F), 0).astype(jnp.float32)
    a = tt[0:_NCP - 1] - ii * d
    acc_a = jnp.broadcast_to(a[0:1], x.shape)
    acc_b = jnp.broadcast_to(d[0:1], x.shape)
    for i in range(1, _NCP - 1):
        m = scaled >= float(i)
        acc_a = jnp.where(m, a[i:i + 1], acc_a)
        acc_b = jnp.where(m, d[i:i + 1], acc_b)
    o_ref[...] = acc_a + scaled * acc_b


_tc_call = pl.pallas_call(
    _tc_body,
    grid=(_N_TC // _BR,),
    in_specs=[
        pl.BlockSpec((8, 128), lambda g: (0, 0)),
        pl.BlockSpec((_NCP, _F), lambda g: (0, 0)),
        pl.BlockSpec((_BR, _F), lambda g: (_R_SC // _BR + g, 0)),
    ],
    out_specs=pl.BlockSpec((_BR, _F), lambda g: (_R_SC // _BR + g, 0)),
    out_shape=jax.ShapeDtypeStruct((_B, _F), jnp.float32),
    input_output_aliases={0: 0},
)


def kernel(x, interp_tensor, feature_idx):
    del feature_idx
    tab = interp_tensor.reshape(-1)
    sc_out = _sc_call(x, tab)
    return _tc_call(sc_out, interp_tensor.T, x)


# speedup vs baseline: 1.4186x; 1.0429x over previous
import jax
import jax.numpy as jnp
from jax import lax
from jax.experimental import pallas as pl
from jax.experimental.pallas import tpu as pltpu, tpu_sc as plsc

_B = 16384
_F = 1024
_NCP = 11
_TAB = _F * _NCP
_LOC = 5.0
_MAXL = 9
_NW = 32
_R_SC = 6144
_RPW = _R_SC // _NW
_CIN = 32
_COUT = 16
_NCIN = _RPW // _CIN
_NCOUT = _RPW // _COUT
_L = 16
_NFB = _F // _L


def _sc_body(x_hbm, tab_hbm, out_hbm, tab_v, pk_v, x_v, o_v,
             sem_in0, sem_in1, sem_out0, sem_out1):
    wid = lax.axis_index("s") * 2 + lax.axis_index("c")
    row0 = wid * _RPW
    pltpu.sync_copy(tab_hbm, tab_v.at[pl.ds(0, _TAB)])
    iota = lax.iota(jnp.int32, _L)
    iota11 = iota * _NCP

    def tab_body(k, carry):
        p0 = k * _L
        tv = tab_v[pl.ds(p0, _L)]
        tv1 = tab_v[pl.ds(p0 + 1, _L)]
        ivec = jnp.mod(p0 + iota, _NCP).astype(jnp.float32)
        d = tv1 - tv
        a = tv - ivec * d
        pk = plsc.bitcast(plsc.pack(a, d, format=plsc.PackFormat.INTERLEAVED),
                          jnp.int32)
        pk_v[pl.ds(p0, _L)] = pk
        return carry

    lax.fori_loop(0, _TAB // _L, tab_body, 0)

    sems_in = (sem_in0, sem_in1)
    sems_out = (sem_out0, sem_out1)

    def start_in(c, b):
        pltpu.async_copy(x_hbm.at[pl.ds(row0 + c * _CIN, _CIN)],
                         x_v.at[b], sems_in[b]).start()

    def wait_in(b):
        pltpu.make_async_copy(x_hbm.at[pl.ds(0, _CIN)], x_v.at[b],
                              sems_in[b]).wait()

    def start_out(oh, ob):
        pltpu.async_copy(o_v.at[ob], out_hbm.at[pl.ds(row0 + oh * _COUT, _COUT)],
                         sems_out[ob]).start()

    def wait_out(ob):
        pltpu.make_async_copy(o_v.at[ob], out_hbm.at[pl.ds(0, _COUT)],
                              sems_out[ob]).wait()

    start_in(0, 0)
    start_in(1, 1)

    def compute_half(b, h, ob):
        def fb_body(j, carry):
            f0 = j * _L
            fbase = f0 * _NCP + iota11

            @plsc.parallel_loop(0, _COUT, unroll=4)
            def row_body(r):
                xv = x_v[b, h * _COUT + r, pl.ds(f0, _L)]
                scaled = xv + _LOC
                li = jnp.clip(scaled.astype(jnp.int32), 0, _MAXL)
                flat = fbase + li
                pk = plsc.load_gather(pk_v, [flat])
                ab = plsc.bitcast(pk, jnp.bfloat16)
                av, bv = plsc.unpack(ab, format=plsc.PackFormat.INTERLEAVED)
                o_v[ob, r, pl.ds(f0, _L)] = av + scaled * bv

            return carry

        lax.fori_loop(0, _NFB, fb_body, 0)

    def cc_body(cc, carry):
        for b in range(2):
            c = cc * 2 + b
            wait_in(b)
            for h in range(2):
                oh = 2 * c + h

                @pl.when(oh >= 2)
                def _():
                    wait_out(h)

                compute_half(b, h, h)
                start_out(oh, h)

            @pl.when(c + 2 < _NCIN)
            def _():
                start_in(c + 2, b)

        return carry

    lax.fori_loop(0, _NCIN // 2, cc_body, 0)
    wait_out(0)
    wait_out(1)


_sc_call = pl.kernel(
    _sc_body,
    out_type=jax.ShapeDtypeStruct((_B, _F), jnp.float32),
    mesh=plsc.VectorSubcoreMesh(core_axis_name="c", subcore_axis_name="s"),
    compiler_params=pltpu.CompilerParams(needs_layout_passes=False),
    scratch_types=[
        pltpu.VMEM((_TAB + _L,), jnp.float32),
        pltpu.VMEM((_TAB,), jnp.int32),
        pltpu.VMEM((2, _CIN, _F), jnp.float32),
        pltpu.VMEM((2, _COUT, _F), jnp.float32),
        pltpu.SemaphoreType.DMA,
        pltpu.SemaphoreType.DMA,
        pltpu.SemaphoreType.DMA,
        pltpu.SemaphoreType.DMA,
    ],
)


_BR = 512
_N_TC = _B - _R_SC


def _tc_body(sc_ref, tt_ref, x_ref, o_ref):
    del sc_ref
    tt = tt_ref[...]
    x = x_ref[...]
    scaled = x + _LOC
    d = tt[1:_NCP] - tt[0:_NCP - 1]
    ii = lax.broadcasted_iota(jnp.int32, (_NCP - 1, _F), 0).astype(jnp.float32)
    a = tt[0:_NCP - 1] - ii * d
    acc_a = jnp.broadcast_to(a[0:1], x.shape)
    acc_b = jnp.broadcast_to(d[0:1], x.shape)
    for i in range(1, _NCP - 1):
        m = scaled >= float(i)
        acc_a = jnp.where(m, a[i:i + 1], acc_a)
        acc_b = jnp.where(m, d[i:i + 1], acc_b)
    o_ref[...] = acc_a + scaled * acc_b


_tc_call = pl.pallas_call(
    _tc_body,
    grid=(_N_TC // _BR,),
    in_specs=[
        pl.BlockSpec((8, 128), lambda g: (0, 0)),
        pl.BlockSpec((_NCP, _F), lambda g: (0, 0)),
        pl.BlockSpec((_BR, _F), lambda g: (_R_SC // _BR + g, 0)),
    ],
    out_specs=pl.BlockSpec((_BR, _F), lambda g: (_R_SC // _BR + g, 0)),
    out_shape=jax.ShapeDtypeStruct((_B, _F), jnp.float32),
    input_output_aliases={0: 0},
)


def kernel(x, interp_tensor, feature_idx):
    del feature_idx
    tab = interp_tensor.reshape(-1)
    sc_out = _sc_call(x, tab)
    return _tc_call(sc_out, interp_tensor.T, x)
